# baseline (speedup 1.0000x reference)
import functools

import jax
import jax.numpy as jnp
from jax import lax
from jax.experimental import pallas as pl
from jax.experimental.pallas import tpu as pltpu
from jax.experimental.pallas import tpu_sc as plsc

_NC = 2
_NS = 16
_NW = _NC * _NS
_LANES = 16


@functools.lru_cache(maxsize=None)
def _make_lookup(n_tokens: int, vocab: int, d_model: int):
    assert d_model == 128
    assert n_tokens % (_NW * 256) == 0
    per_w = n_tokens // _NW
    chunk = 256
    n_chunks = per_w // chunk
    groups = chunk // _LANES

    mesh = plsc.VectorSubcoreMesh(core_axis_name="c", subcore_axis_name="s")

    @functools.partial(
        pl.kernel,
        mesh=mesh,
        out_type=jax.ShapeDtypeStruct((n_tokens * d_model,), jnp.float32),
        scratch_types=[
            pltpu.VMEM((vocab * d_model,), jnp.float32),
            pltpu.VMEM((chunk,), jnp.int32),
            pltpu.VMEM((chunk * d_model,), jnp.float32),
        ],
        compiler_params=pltpu.CompilerParams(needs_layout_passes=False),
    )
    def lookup(tok_hbm, tab_hbm, out_hbm, tab_v, idx_v, out_v):
        wid = lax.axis_index("s") * _NC + lax.axis_index("c")
        base = wid * per_w
        pltpu.sync_copy(tab_hbm, tab_v)
        iota = lax.iota(jnp.int32, 16)
        iota_d = iota * d_model

        def chunk_body(g, carry):
            tbase = base + g * chunk
            pltpu.sync_copy(tok_hbm.at[pl.ds(tbase, chunk)], idx_v)

            def group_body(g2, carry2):
                toks = idx_v[pl.ds(g2 * _LANES, _LANES)]
                src_base = toks * d_model
                dst_base = iota_d + g2 * (_LANES * d_model)
                for c in range(d_model):
                    vals = plsc.load_gather(tab_v, [src_base + c])
                    plsc.store_scatter(out_v, [dst_base + c], vals)
                return carry2

            lax.fori_loop(0, groups, group_body, 0, unroll=False)
            pltpu.sync_copy(out_v, out_hbm.at[pl.ds(tbase * d_model, chunk * d_model)])
            return carry

        lax.fori_loop(0, n_chunks, chunk_body, 0, unroll=False)

    return lookup


def kernel(tokens, embed_weight):
    b, k, l = tokens.shape
    vocab, d_model = embed_weight.shape
    n = b * k * l
    tok_flat = tokens.reshape((n,)).astype(jnp.int32)
    tab_flat = embed_weight.reshape((vocab * d_model,))
    out = _make_lookup(n, vocab, d_model)(tok_flat, tab_flat)
    return out.reshape((b, k, l, d_model))


# speedup vs baseline: 2.3898x; 2.3898x over previous
import functools

import jax
import jax.numpy as jnp
from jax import lax
from jax.experimental import pallas as pl
from jax.experimental.pallas import tpu as pltpu
from jax.experimental.pallas import tpu_sc as plsc

_NC = 2
_NS = 16
_NW = _NC * _NS
_CHUNK = 256
_IW = 128


@functools.lru_cache(maxsize=None)
def _make_lookup(n_tokens: int, vocab: int, d_model: int):
    assert n_tokens % (_NW * _CHUNK) == 0
    per_w = n_tokens // _NW
    n_chunks = per_w // _CHUNK
    n_steps = n_chunks // 2
    tok_rows = per_w // _IW

    mesh = plsc.VectorSubcoreMesh(core_axis_name="c", subcore_axis_name="s")

    @functools.partial(
        pl.kernel,
        mesh=mesh,
        out_type=jax.ShapeDtypeStruct((n_tokens, d_model), jnp.float32),
        scratch_types=[
            pltpu.VMEM((tok_rows, _IW), jnp.int32),
            pltpu.VMEM((2, _CHUNK, d_model), jnp.float32),
            pltpu.SemaphoreType.DMA,
            pltpu.SemaphoreType.DMA,
            pltpu.SemaphoreType.DMA,
        ],
        compiler_params=pltpu.CompilerParams(needs_layout_passes=False),
    )
    def lookup(tok_hbm, tab_hbm, out_hbm, tok_v, rows_v, sem_g, sem_o0, sem_o1):
        wid = lax.axis_index("s") * _NC + lax.axis_index("c")
        row_base = wid * per_w
        pltpu.sync_copy(tok_hbm.at[pl.ds(wid * tok_rows, tok_rows)], tok_v)
        sems = (sem_o0, sem_o1)

        def do_chunk(g, par):
            gathers = []
            for j in range(_CHUNK // _IW):
                idx = tok_v.at[g * (_CHUNK // _IW) + j]
                dst = rows_v.at[par, pl.ds(j * _IW, _IW)]
                cp = pltpu.make_async_copy(tab_hbm.at[idx], dst, sem_g)
                cp.start()
                gathers.append(cp)
            for cp in gathers:
                cp.wait()
            pltpu.make_async_copy(
                rows_v.at[par],
                out_hbm.at[pl.ds(row_base + g * _CHUNK, _CHUNK)],
                sems[par],
            ).start()

        def out_wait(g, par):
            pltpu.make_async_copy(
                rows_v.at[par],
                out_hbm.at[pl.ds(row_base + g * _CHUNK, _CHUNK)],
                sems[par],
            ).wait()

        do_chunk(0, 0)
        do_chunk(1, 1)

        def step_body(s, carry):
            g = 2 * s
            out_wait(g - 2, 0)
            do_chunk(g, 0)
            out_wait(g - 1, 1)
            do_chunk(g + 1, 1)
            return carry

        lax.fori_loop(1, n_steps, step_body, 0, unroll=False)
        out_wait(n_chunks - 2, 0)
        out_wait(n_chunks - 1, 1)

    return lookup


def kernel(tokens, embed_weight):
    b, k, l = tokens.shape
    vocab, d_model = embed_weight.shape
    n = b * k * l
    tok_2d = tokens.reshape((n // _IW, _IW)).astype(jnp.int32)
    out = _make_lookup(n, vocab, d_model)(tok_2d, embed_weight)
    return out.reshape((b, k, l, d_model))


# speedup vs baseline: 20.2353x; 8.4675x over previous
import functools

import jax
import jax.numpy as jnp
from jax import lax
from jax.experimental import pallas as pl
from jax.experimental.pallas import tpu as pltpu
from jax.experimental.pallas import tpu_sc as plsc

_NC = 2
_NS = 16
_NW = _NC * _NS
_CHUNK = 256
_IW = 128


@functools.lru_cache(maxsize=None)
def _make_lookup(n_tokens: int, vocab: int, d_model: int):
    assert n_tokens % (_NW * _CHUNK) == 0
    per_w = n_tokens // _NW
    n_chunks = per_w // _CHUNK
    n_steps = n_chunks // 2
    tok_rows = per_w // _IW

    mesh = plsc.VectorSubcoreMesh(core_axis_name="c", subcore_axis_name="s")

    @functools.partial(
        pl.kernel,
        mesh=mesh,
        out_type=jax.ShapeDtypeStruct((n_tokens, d_model), jnp.float32),
        scratch_types=[
            pltpu.VMEM_SHARED((vocab, d_model), jnp.float32),
            pltpu.VMEM((tok_rows, _IW), jnp.int32),
            pltpu.VMEM((2, _CHUNK, d_model), jnp.float32),
            pltpu.SemaphoreType.DMA,
            pltpu.SemaphoreType.DMA,
            pltpu.SemaphoreType.DMA,
        ],
        compiler_params=pltpu.CompilerParams(needs_layout_passes=False),
    )
    def lookup(tok_hbm, tab_hbm, out_hbm, tab_v, tok_v, rows_v, sem_g, sem_o0, sem_o1):
        wid = lax.axis_index("s") * _NC + lax.axis_index("c")
        row_base = wid * per_w
        @pl.when(lax.axis_index("s") == 0)
        def _():
            pltpu.sync_copy(tab_hbm, tab_v)

        pltpu.sync_copy(tok_hbm.at[pl.ds(wid * tok_rows, tok_rows)], tok_v)
        plsc.subcore_barrier()
        sems = (sem_o0, sem_o1)

        def do_chunk(g, par):
            gathers = []
            for j in range(_CHUNK // _IW):
                idx = tok_v.at[g * (_CHUNK // _IW) + j]
                dst = rows_v.at[par, pl.ds(j * _IW, _IW)]
                cp = pltpu.make_async_copy(tab_v.at[idx], dst, sem_g)
                cp.start()
                gathers.append(cp)
            for cp in gathers:
                cp.wait()
            pltpu.make_async_copy(
                rows_v.at[par],
                out_hbm.at[pl.ds(row_base + g * _CHUNK, _CHUNK)],
                sems[par],
            ).start()

        def out_wait(g, par):
            pltpu.make_async_copy(
                rows_v.at[par],
                out_hbm.at[pl.ds(row_base + g * _CHUNK, _CHUNK)],
                sems[par],
            ).wait()

        do_chunk(0, 0)
        do_chunk(1, 1)

        def step_body(s, carry):
            g = 2 * s
            out_wait(g - 2, 0)
            do_chunk(g, 0)
            out_wait(g - 1, 1)
            do_chunk(g + 1, 1)
            return carry

        lax.fori_loop(1, n_steps, step_body, 0, unroll=False)
        out_wait(n_chunks - 2, 0)
        out_wait(n_chunks - 1, 1)

    return lookup


def kernel(tokens, embed_weight):
    b, k, l = tokens.shape
    vocab, d_model = embed_weight.shape
    n = b * k * l
    tok_2d = tokens.reshape((n // _IW, _IW)).astype(jnp.int32)
    out = _make_lookup(n, vocab, d_model)(tok_2d, embed_weight)
    return out.reshape((b, k, l, d_model))


# speedup vs baseline: 20.4718x; 1.0117x over previous
import functools

import jax
import jax.numpy as jnp
from jax import lax
from jax.experimental import pallas as pl
from jax.experimental.pallas import tpu as pltpu
from jax.experimental.pallas import tpu_sc as plsc

_NC = 2
_NS = 16
_NW = _NC * _NS
_CHUNK = 256
_IW = 128


@functools.lru_cache(maxsize=None)
def _make_lookup(n_tokens: int, vocab: int, d_model: int):
    assert n_tokens % (_NW * _CHUNK) == 0
    per_w = n_tokens // _NW
    n_chunks = per_w // _CHUNK
    n_steps = n_chunks // 2
    tok_rows = per_w // _IW

    mesh = plsc.VectorSubcoreMesh(core_axis_name="c", subcore_axis_name="s")

    @functools.partial(
        pl.kernel,
        mesh=mesh,
        out_type=jax.ShapeDtypeStruct((n_tokens, d_model), jnp.float32),
        scratch_types=[
            pltpu.VMEM_SHARED((vocab, d_model), jnp.float32),
            pltpu.VMEM((tok_rows, _IW), jnp.int32),
            pltpu.VMEM((2, _CHUNK, d_model), jnp.float32),
            pltpu.SemaphoreType.DMA,
            pltpu.SemaphoreType.DMA,
            pltpu.SemaphoreType.DMA,
        ],
        compiler_params=pltpu.CompilerParams(needs_layout_passes=False),
    )
    def lookup(tok_hbm, tab_hbm, out_hbm, tab_v, tok_v, rows_v, sem_g, sem_o0, sem_o1):
        wid = lax.axis_index("s") * _NC + lax.axis_index("c")
        row_base = wid * per_w
        @pl.when(lax.axis_index("s") == 0)
        def _():
            pltpu.sync_copy(tab_hbm, tab_v)

        pltpu.sync_copy(tok_hbm.at[pl.ds(wid * tok_rows, tok_rows)], tok_v)
        plsc.subcore_barrier()
        sems = (sem_o0, sem_o1)

        def do_chunk(g, par):
            gathers = []
            for j in range(_CHUNK // _IW):
                idx = tok_v.at[g * (_CHUNK // _IW) + j]
                dst = rows_v.at[par, pl.ds(j * _IW, _IW)]
                cp = pltpu.make_async_copy(tab_v.at[idx], dst, sem_g)
                cp.start()
                gathers.append(cp)
            for j, cp in enumerate(gathers):
                cp.wait()
                pltpu.make_async_copy(
                    rows_v.at[par, pl.ds(j * _IW, _IW)],
                    out_hbm.at[pl.ds(row_base + g * _CHUNK + j * _IW, _IW)],
                    sems[par],
                ).start()

        def out_wait(g, par):
            pltpu.make_async_copy(
                rows_v.at[par],
                out_hbm.at[pl.ds(row_base + g * _CHUNK, _CHUNK)],
                sems[par],
            ).wait()

        do_chunk(0, 0)
        do_chunk(1, 1)

        def step_body(s, carry):
            g = 2 * s
            out_wait(g - 2, 0)
            do_chunk(g, 0)
            out_wait(g - 1, 1)
            do_chunk(g + 1, 1)
            return carry

        lax.fori_loop(1, n_steps, step_body, 0, unroll=False)
        out_wait(n_chunks - 2, 0)
        out_wait(n_chunks - 1, 1)

    return lookup


def kernel(tokens, embed_weight):
    b, k, l = tokens.shape
    vocab, d_model = embed_weight.shape
    n = b * k * l
    tok_2d = tokens.reshape((n // _IW, _IW)).astype(jnp.int32)
    out = _make_lookup(n, vocab, d_model)(tok_2d, embed_weight)
    return out.reshape((b, k, l, d_model))


# speedup vs baseline: 20.7072x; 1.0115x over previous
import functools

import jax
import jax.numpy as jnp
from jax import lax
from jax.experimental import pallas as pl
from jax.experimental.pallas import tpu as pltpu
from jax.experimental.pallas import tpu_sc as plsc

_NC = 2
_NS = 16
_NW = _NC * _NS
_CHUNK = 256
_IW = 128


_RING = 4


@functools.lru_cache(maxsize=None)
def _make_lookup(n_tokens: int, vocab: int, d_model: int):
    assert n_tokens % (_NW * _IW * _RING) == 0
    per_w = n_tokens // _NW
    n_halves = per_w // _IW
    n_steps = n_halves // _RING
    tok_rows = per_w // _IW

    mesh = plsc.VectorSubcoreMesh(core_axis_name="c", subcore_axis_name="s")

    @functools.partial(
        pl.kernel,
        mesh=mesh,
        out_type=jax.ShapeDtypeStruct((n_tokens, d_model), jnp.float32),
        scratch_types=[
            pltpu.VMEM_SHARED((vocab, d_model), jnp.float32),
            pltpu.VMEM((tok_rows, _IW), jnp.int32),
            pltpu.VMEM((_RING, _IW, d_model), jnp.float32),
            pltpu.SemaphoreType.DMA,
        ] + [pltpu.SemaphoreType.DMA] * _RING,
        compiler_params=pltpu.CompilerParams(needs_layout_passes=False),
    )
    def lookup(tok_hbm, tab_hbm, out_hbm, tab_v, tok_v, rows_v, sem_g, *sem_o):
        wid = lax.axis_index("s") * _NC + lax.axis_index("c")
        row_base = wid * per_w
        @pl.when(lax.axis_index("s") == 0)
        def _():
            pltpu.sync_copy(tab_hbm, tab_v)

        pltpu.sync_copy(tok_hbm.at[pl.ds(wid * tok_rows, tok_rows)], tok_v)
        plsc.subcore_barrier()

        def gather_start(h, sl):
            return pltpu.make_async_copy(
                tab_v.at[tok_v.at[h]], rows_v.at[sl], sem_g)

        def out_copy(h, sl):
            return pltpu.make_async_copy(
                rows_v.at[sl],
                out_hbm.at[pl.ds(row_base + h * _IW, _IW)],
                sem_o[sl],
            )

        def run_step(h0, first):
            gathers = []
            for sl in range(_RING):
                if not first:
                    out_copy(h0 + sl - _RING, sl).wait()
                cp = gather_start(h0 + sl, sl)
                cp.start()
                gathers.append(cp)
            for sl in range(_RING):
                gathers[sl].wait()
                out_copy(h0 + sl, sl).start()

        run_step(0, True)
        lax.fori_loop(
            1, n_steps,
            lambda s, c: (run_step(s * _RING, False), c)[1], 0,
            unroll=False)
        for sl in range(_RING):
            out_copy(n_halves - _RING + sl, sl).wait()

    return lookup


def kernel(tokens, embed_weight):
    b, k, l = tokens.shape
    vocab, d_model = embed_weight.shape
    n = b * k * l
    tok_2d = tokens.reshape((n // _IW, _IW)).astype(jnp.int32)
    out = _make_lookup(n, vocab, d_model)(tok_2d, embed_weight)
    return out.reshape((b, k, l, d_model))
